# Initial kernel scaffold; baseline (speedup 1.0000x reference)
import jax
import jax.numpy as jnp
from jax.experimental import pallas as pl


def kernel(x, edge_index, W1, b1, W2, b2):
    raise NotImplementedError("write your pallas kernel here")


# speedup vs baseline: 3.7851x; 3.7851x over previous
import functools

import jax
import jax.numpy as jnp
from jax import lax
from jax.experimental import pallas as pl
from jax.experimental.pallas import tpu as pltpu
from jax.experimental.pallas import tpu_sc as plsc

K_HOPS = 10
ALPHA = 0.1
CH = 128
ROWB = 128


def _sc_geometry():
  info = plsc.get_sparse_core_info()
  return info.num_cores, info.num_subcores


DW = 16


def _make_count(NW, NCHT, N_pad):
  NC, NS = _sc_geometry()
  mesh = plsc.VectorSubcoreMesh(core_axis_name="c", subcore_axis_name="s")
  STRIPE = N_pad // NS

  @functools.partial(
      pl.kernel,
      out_type=jax.ShapeDtypeStruct((NC, N_pad, DW), jnp.float32),
      mesh=mesh,
      scratch_types=[
          pltpu.VMEM_SHARED((N_pad, DW), jnp.float32),
          pltpu.VMEM((STRIPE, DW), jnp.float32),
          pltpu.VMEM((CH,), jnp.int32),
          pltpu.VMEM((CH, DW), jnp.float32),
      ],
  )
  def count(dst_hbm, cnt_hbm, acc, zbuf, dstb, ones_rows):
    c = lax.axis_index("c")
    s = lax.axis_index("s")
    w = c * NS + s
    zero16 = jnp.zeros((16,), jnp.float32)
    ones16 = jnp.ones((16,), jnp.float32)

    def zbody(i, _):
      zbuf[i, :] = zero16
      return 0
    lax.fori_loop(0, STRIPE, zbody, 0)
    pltpu.sync_copy(zbuf, acc.at[pl.ds(s * STRIPE, STRIPE), :])

    def obody(i, _):
      ones_rows[i, :] = ones16
      return 0
    lax.fori_loop(0, CH, obody, 0)
    plsc.subcore_barrier()

    def ebody(j, _):
      pltpu.sync_copy(dst_hbm.at[w, j], dstb)
      pltpu.sync_copy(ones_rows, acc.at[dstb], add=True)
      return 0
    lax.fori_loop(0, NCHT, ebody, 0)
    plsc.subcore_barrier()
    pltpu.sync_copy(acc.at[pl.ds(s * STRIPE, STRIPE), :],
                    cnt_hbm.at[c, pl.ds(s * STRIPE, STRIPE), :])

  return count


def _make_hop_scatter(NW, NCHT, N_pad, F_OUT):
  NC, NS = _sc_geometry()
  mesh = plsc.VectorSubcoreMesh(core_axis_name="c", subcore_axis_name="s")
  STRIPE = N_pad // NS
  ZR = STRIPE // 8

  @functools.partial(
      pl.kernel,
      out_type=jax.ShapeDtypeStruct((NC, N_pad, F_OUT), jnp.float32),
      mesh=mesh,
      scratch_types=[
          pltpu.VMEM_SHARED((N_pad, F_OUT), jnp.float32),
          pltpu.VMEM((ZR, F_OUT), jnp.float32),
          pltpu.VMEM((CH,), jnp.int32),
          pltpu.VMEM((CH,), jnp.int32),
          pltpu.VMEM((CH, F_OUT), jnp.float32),
          pltpu.SemaphoreType.DMA,
      ],
  )
  def hop(g_hbm, src_hbm, dst_hbm, s_hbm, acc, zbuf, srcb, dstb, rows, sem):
    c = lax.axis_index("c")
    s = lax.axis_index("s")
    w = c * NS + s
    zero16 = jnp.zeros((16,), jnp.float32)

    def zbody(i, _):
      r = i // (F_OUT // 16)
      k = i % (F_OUT // 16)
      zbuf[r, pl.ds(k * 16, 16)] = zero16
      return 0
    lax.fori_loop(0, ZR * (F_OUT // 16), zbody, 0)
    for i in range(8):
      pltpu.sync_copy(zbuf, acc.at[pl.ds(s * STRIPE + i * ZR, ZR), :])
    plsc.subcore_barrier()

    def ebody(j, _):
      pltpu.sync_copy(src_hbm.at[w, j], srcb)
      pltpu.sync_copy(dst_hbm.at[w, j], dstb)
      pltpu.async_copy(g_hbm.at[srcb], rows, sem).wait()
      pltpu.sync_copy(rows, acc.at[dstb], add=True)
      return 0
    lax.fori_loop(0, NCHT, ebody, 0)
    plsc.subcore_barrier()
    pltpu.sync_copy(acc.at[pl.ds(s * STRIPE, STRIPE), :],
                    s_hbm.at[c, pl.ds(s * STRIPE, STRIPE), :])

  return hop


def _mlp_body(x_ref, w1_ref, b1_ref, w2_ref, b2_ref, h0_ref):
  t = jnp.dot(x_ref[...], w1_ref[...], preferred_element_type=jnp.float32)
  t = jnp.maximum(t + b1_ref[...], 0.0)
  h0_ref[...] = (
      jnp.dot(t, w2_ref[...], preferred_element_type=jnp.float32) + b2_ref[...]
  )


def _prep_body(cnt_ref, h0_ref, g0_ref, dinv_ref):
  deg = 1.0 + jnp.sum(cnt_ref[...], axis=(0, 2)) * (1.0 / DW)
  dinv = lax.rsqrt(deg)[:, None]
  dinv_ref[...] = dinv
  g0_ref[...] = h0_ref[...] * dinv


def _update_body(s_ref, g_ref, h0_ref, dinv_ref, gn_ref):
  agg = s_ref[0] + s_ref[1] + g_ref[...]
  dinv = dinv_ref[...]
  h_new = ALPHA * h0_ref[...] + (1.0 - ALPHA) * (dinv * agg)
  gn_ref[...] = dinv * h_new


def _update_last_body(s_ref, g_ref, h0_ref, dinv_ref, out_ref):
  agg = s_ref[0] + s_ref[1] + g_ref[...]
  dinv = dinv_ref[...]
  h_new = ALPHA * h0_ref[...] + (1.0 - ALPHA) * (dinv * agg)
  m = jnp.max(h_new, axis=1, keepdims=True)
  z = h_new - m
  lse = jnp.log(jnp.sum(jnp.exp(z), axis=1, keepdims=True))
  out_ref[...] = z - lse


def kernel(x, edge_index, W1, b1, W2, b2):
  N, F_IN = x.shape
  F_HID = W1.shape[1]
  F_OUT = W2.shape[1]
  E = edge_index.shape[1]
  NC, NS = _sc_geometry()
  NW = NC * NS

  N_pad = ((N + 1 + ROWB - 1) // ROWB) * ROWB
  NCHT = -(-E // (NW * CH))
  E_pad = NW * CH * NCHT

  src = edge_index[0].astype(jnp.int32)
  dst = edge_index[1].astype(jnp.int32)
  fill = jnp.full((E_pad - E,), N, jnp.int32)
  src_p = jnp.concatenate([src, fill]).reshape(NW, NCHT, CH)
  dst_p = jnp.concatenate([dst, fill]).reshape(NW, NCHT, CH)
  x_p = jnp.zeros((N_pad, F_IN), x.dtype).at[:N].set(x)
  b1r = b1.reshape(1, F_HID)
  b2r = b2.reshape(1, F_OUT)

  grid = (N_pad // ROWB,)

  cnt = _make_count(NW, NCHT, N_pad)(dst_p)

  h0 = pl.pallas_call(
      _mlp_body,
      grid=grid,
      in_specs=[
          pl.BlockSpec((ROWB, F_IN), lambda i: (i, 0)),
          pl.BlockSpec((F_IN, F_HID), lambda i: (0, 0)),
          pl.BlockSpec((1, F_HID), lambda i: (0, 0)),
          pl.BlockSpec((F_HID, F_OUT), lambda i: (0, 0)),
          pl.BlockSpec((1, F_OUT), lambda i: (0, 0)),
      ],
      out_specs=pl.BlockSpec((ROWB, F_OUT), lambda i: (i, 0)),
      out_shape=jax.ShapeDtypeStruct((N_pad, F_OUT), jnp.float32),
  )(x_p, W1, b1r, W2, b2r)

  g, dinv = pl.pallas_call(
      _prep_body,
      grid=grid,
      in_specs=[
          pl.BlockSpec((NC, ROWB, DW), lambda i: (0, i, 0)),
          pl.BlockSpec((ROWB, F_OUT), lambda i: (i, 0)),
      ],
      out_specs=[
          pl.BlockSpec((ROWB, F_OUT), lambda i: (i, 0)),
          pl.BlockSpec((ROWB, 1), lambda i: (i, 0)),
      ],
      out_shape=[
          jax.ShapeDtypeStruct((N_pad, F_OUT), jnp.float32),
          jax.ShapeDtypeStruct((N_pad, 1), jnp.float32),
      ],
  )(cnt, h0)

  hop = _make_hop_scatter(NW, NCHT, N_pad, F_OUT)
  upd_specs = dict(
      grid=grid,
      in_specs=[
          pl.BlockSpec((NC, ROWB, F_OUT), lambda i: (0, i, 0)),
          pl.BlockSpec((ROWB, F_OUT), lambda i: (i, 0)),
          pl.BlockSpec((ROWB, F_OUT), lambda i: (i, 0)),
          pl.BlockSpec((ROWB, 1), lambda i: (i, 0)),
      ],
      out_specs=pl.BlockSpec((ROWB, F_OUT), lambda i: (i, 0)),
      out_shape=jax.ShapeDtypeStruct((N_pad, F_OUT), jnp.float32),
  )

  for k in range(K_HOPS):
    S = hop(g, src_p, dst_p)
    body = _update_last_body if k == K_HOPS - 1 else _update_body
    g = pl.pallas_call(body, **upd_specs)(S, g, h0, dinv)

  return g[:N]


# speedup vs baseline: 4.5932x; 1.2135x over previous
import functools

import jax
import jax.numpy as jnp
from jax import lax
from jax.experimental import pallas as pl
from jax.experimental.pallas import tpu as pltpu
from jax.experimental.pallas import tpu_sc as plsc

K_HOPS = 10
ALPHA = 0.1
CH = 128
ROWB = 128


def _sc_geometry():
  info = plsc.get_sparse_core_info()
  return info.num_cores, info.num_subcores


DW = 16


def _make_count(NW, NCHT, N_pad):
  NC, NS = _sc_geometry()
  mesh = plsc.VectorSubcoreMesh(core_axis_name="c", subcore_axis_name="s")
  STRIPE = N_pad // NS

  NB = 4

  @functools.partial(
      pl.kernel,
      out_type=jax.ShapeDtypeStruct((NC, N_pad, DW), jnp.float32),
      mesh=mesh,
      scratch_types=[
          pltpu.VMEM_SHARED((N_pad, DW), jnp.float32),
          pltpu.VMEM((STRIPE, DW), jnp.float32),
          pltpu.VMEM((NCHT, CH), jnp.int32),
          pltpu.VMEM((CH, DW), jnp.float32),
      ] + [pltpu.SemaphoreType.DMA for _ in range(NB)],
  )
  def count(dst_hbm, cnt_hbm, acc, zbuf, dstv, ones_rows, *ssem):
    c = lax.axis_index("c")
    s = lax.axis_index("s")
    w = c * NS + s
    zero16 = jnp.zeros((16,), jnp.float32)
    ones16 = jnp.ones((16,), jnp.float32)

    pltpu.sync_copy(dst_hbm.at[w], dstv)

    def zbody(i, _):
      zbuf[i, :] = zero16
      return 0
    lax.fori_loop(0, STRIPE, zbody, 0)
    pltpu.sync_copy(zbuf, acc.at[pl.ds(s * STRIPE, STRIPE), :])

    def obody(i, _):
      ones_rows[i, :] = ones16
      return 0
    lax.fori_loop(0, CH, obody, 0)
    plsc.subcore_barrier()

    sd = [None] * NCHT
    for j in range(NCHT):
      if j >= NB:
        sd[j - NB].wait()
      sd[j] = pltpu.async_copy(ones_rows, acc.at[dstv.at[j]],
                               ssem[j % NB], add=True)
    for j in range(max(0, NCHT - NB), NCHT):
      sd[j].wait()
    plsc.subcore_barrier()
    pltpu.sync_copy(acc.at[pl.ds(s * STRIPE, STRIPE), :],
                    cnt_hbm.at[c, pl.ds(s * STRIPE, STRIPE), :])

  return count


def _make_hop_scatter(NW, NCHT, N_pad, F_OUT):
  NC, NS = _sc_geometry()
  mesh = plsc.VectorSubcoreMesh(core_axis_name="c", subcore_axis_name="s")
  STRIPE = N_pad // NS

  NB = 2

  @functools.partial(
      pl.kernel,
      out_type=jax.ShapeDtypeStruct((NC, N_pad, F_OUT), jnp.float32),
      mesh=mesh,
      scratch_types=[
          pltpu.VMEM_SHARED((N_pad, F_OUT), jnp.float32),
          pltpu.VMEM((NCHT, CH), jnp.int32),
          pltpu.VMEM((NCHT, CH), jnp.int32),
      ]
      + [pltpu.VMEM((CH, F_OUT), jnp.float32) for _ in range(NB)]
      + [pltpu.SemaphoreType.DMA for _ in range(2 * NB)],
  )
  def hop(g_hbm, src_hbm, dst_hbm, s_hbm, acc, srcv, dstv, *bufsem):
    rows = bufsem[:NB]
    gsem = bufsem[NB:2 * NB]
    ssem = bufsem[2 * NB:]
    c = lax.axis_index("c")
    s = lax.axis_index("s")
    w = c * NS + s
    zero16 = jnp.zeros((16,), jnp.float32)

    pltpu.sync_copy(src_hbm.at[w], srcv)
    pltpu.sync_copy(dst_hbm.at[w], dstv)

    def zbody(i, _):
      for k in range(F_OUT // 16):
        rows[0][i, pl.ds(k * 16, 16)] = zero16
      return 0
    lax.fori_loop(0, CH, zbody, 0)
    nfull, rem = STRIPE // CH, STRIPE % CH
    for i in range(nfull):
      pltpu.sync_copy(rows[0], acc.at[pl.ds(s * STRIPE + i * CH, CH), :])
    if rem:
      pltpu.sync_copy(rows[0].at[pl.ds(0, rem), :],
                      acc.at[pl.ds(s * STRIPE + nfull * CH, rem), :])
    plsc.subcore_barrier()

    gd = [None] * NCHT
    sd = [None] * NCHT
    for j in range(NCHT):
      b = j % NB
      if j >= NB:
        sd[j - NB].wait()
      gd[j] = pltpu.async_copy(g_hbm.at[srcv.at[j]], rows[b], gsem[b])
      if j >= NB - 1:
        k = j - (NB - 1)
        gd[k].wait()
        sd[k] = pltpu.async_copy(rows[k % NB], acc.at[dstv.at[k]],
                                 ssem[k % NB], add=True)
    for k in range(max(0, NCHT - (NB - 1)), NCHT):
      gd[k].wait()
      sd[k] = pltpu.async_copy(rows[k % NB], acc.at[dstv.at[k]],
                               ssem[k % NB], add=True)
    for k in range(max(0, NCHT - NB), NCHT):
      sd[k].wait()

    plsc.subcore_barrier()
    pltpu.sync_copy(acc.at[pl.ds(s * STRIPE, STRIPE), :],
                    s_hbm.at[c, pl.ds(s * STRIPE, STRIPE), :])

  return hop


def _mlp_body(x_ref, w1_ref, b1_ref, w2_ref, b2_ref, h0_ref):
  t = jnp.dot(x_ref[...], w1_ref[...], preferred_element_type=jnp.float32)
  t = jnp.maximum(t + b1_ref[...], 0.0)
  h0_ref[...] = (
      jnp.dot(t, w2_ref[...], preferred_element_type=jnp.float32) + b2_ref[...]
  )


def _prep_body(cnt_ref, h0_ref, g0_ref, dinv_ref):
  deg = 1.0 + jnp.sum(cnt_ref[...], axis=(0, 2)) * (1.0 / DW)
  dinv = lax.rsqrt(deg)[:, None]
  dinv_ref[...] = dinv
  g0_ref[...] = h0_ref[...] * dinv


def _update_body(s_ref, g_ref, h0_ref, dinv_ref, gn_ref):
  agg = s_ref[0] + s_ref[1] + g_ref[...]
  dinv = dinv_ref[...]
  h_new = ALPHA * h0_ref[...] + (1.0 - ALPHA) * (dinv * agg)
  gn_ref[...] = dinv * h_new


def _update_last_body(s_ref, g_ref, h0_ref, dinv_ref, out_ref):
  agg = s_ref[0] + s_ref[1] + g_ref[...]
  dinv = dinv_ref[...]
  h_new = ALPHA * h0_ref[...] + (1.0 - ALPHA) * (dinv * agg)
  m = jnp.max(h_new, axis=1, keepdims=True)
  z = h_new - m
  lse = jnp.log(jnp.sum(jnp.exp(z), axis=1, keepdims=True))
  out_ref[...] = z - lse


def kernel(x, edge_index, W1, b1, W2, b2):
  N, F_IN = x.shape
  F_HID = W1.shape[1]
  F_OUT = W2.shape[1]
  E = edge_index.shape[1]
  NC, NS = _sc_geometry()
  NW = NC * NS

  N_pad = ((N + 1 + ROWB - 1) // ROWB) * ROWB
  NCHT = -(-E // (NW * CH))
  E_pad = NW * CH * NCHT

  src = edge_index[0].astype(jnp.int32)
  dst = edge_index[1].astype(jnp.int32)
  fill = jnp.full((E_pad - E,), N, jnp.int32)
  src_p = jnp.concatenate([src, fill]).reshape(NW, NCHT, CH)
  dst_p = jnp.concatenate([dst, fill]).reshape(NW, NCHT, CH)
  x_p = jnp.zeros((N_pad, F_IN), x.dtype).at[:N].set(x)
  b1r = b1.reshape(1, F_HID)
  b2r = b2.reshape(1, F_OUT)

  grid = (N_pad // ROWB,)

  cnt = _make_count(NW, NCHT, N_pad)(dst_p)

  h0 = pl.pallas_call(
      _mlp_body,
      grid=grid,
      in_specs=[
          pl.BlockSpec((ROWB, F_IN), lambda i: (i, 0)),
          pl.BlockSpec((F_IN, F_HID), lambda i: (0, 0)),
          pl.BlockSpec((1, F_HID), lambda i: (0, 0)),
          pl.BlockSpec((F_HID, F_OUT), lambda i: (0, 0)),
          pl.BlockSpec((1, F_OUT), lambda i: (0, 0)),
      ],
      out_specs=pl.BlockSpec((ROWB, F_OUT), lambda i: (i, 0)),
      out_shape=jax.ShapeDtypeStruct((N_pad, F_OUT), jnp.float32),
  )(x_p, W1, b1r, W2, b2r)

  g, dinv = pl.pallas_call(
      _prep_body,
      grid=grid,
      in_specs=[
          pl.BlockSpec((NC, ROWB, DW), lambda i: (0, i, 0)),
          pl.BlockSpec((ROWB, F_OUT), lambda i: (i, 0)),
      ],
      out_specs=[
          pl.BlockSpec((ROWB, F_OUT), lambda i: (i, 0)),
          pl.BlockSpec((ROWB, 1), lambda i: (i, 0)),
      ],
      out_shape=[
          jax.ShapeDtypeStruct((N_pad, F_OUT), jnp.float32),
          jax.ShapeDtypeStruct((N_pad, 1), jnp.float32),
      ],
  )(cnt, h0)

  hop = _make_hop_scatter(NW, NCHT, N_pad, F_OUT)
  upd_specs = dict(
      grid=grid,
      in_specs=[
          pl.BlockSpec((NC, ROWB, F_OUT), lambda i: (0, i, 0)),
          pl.BlockSpec((ROWB, F_OUT), lambda i: (i, 0)),
          pl.BlockSpec((ROWB, F_OUT), lambda i: (i, 0)),
          pl.BlockSpec((ROWB, 1), lambda i: (i, 0)),
      ],
      out_specs=pl.BlockSpec((ROWB, F_OUT), lambda i: (i, 0)),
      out_shape=jax.ShapeDtypeStruct((N_pad, F_OUT), jnp.float32),
  )

  for k in range(K_HOPS):
    S = hop(g, src_p, dst_p)
    body = _update_last_body if k == K_HOPS - 1 else _update_body
    g = pl.pallas_call(body, **upd_specs)(S, g, h0, dinv)

  return g[:N]


# speedup vs baseline: 11.0769x; 2.4116x over previous
import functools

import jax
import jax.numpy as jnp
from jax import lax
from jax.experimental import pallas as pl
from jax.experimental.pallas import tpu as pltpu
from jax.experimental.pallas import tpu_sc as plsc

K_HOPS = 10
ALPHA = 0.1
CH = 128
ROWB = 128


def _sc_geometry():
  info = plsc.get_sparse_core_info()
  return info.num_cores, info.num_subcores


DW = 16


def _make_count(NW, NCHT, N_pad):
  NC, NS = _sc_geometry()
  mesh = plsc.VectorSubcoreMesh(core_axis_name="c", subcore_axis_name="s")
  STRIPE = N_pad // NS

  NB = 4

  @functools.partial(
      pl.kernel,
      out_type=jax.ShapeDtypeStruct((NC, N_pad, DW), jnp.float32),
      mesh=mesh,
      scratch_types=[
          pltpu.VMEM_SHARED((N_pad, DW), jnp.float32),
          pltpu.VMEM((STRIPE, DW), jnp.float32),
          pltpu.VMEM((NCHT, 2, CH), jnp.int32),
          pltpu.VMEM((CH, DW), jnp.float32),
      ] + [pltpu.SemaphoreType.DMA for _ in range(NB)],
  )
  def count(edge_hbm, cnt_hbm, acc, zbuf, dstv, ones_rows, *ssem):
    c = lax.axis_index("c")
    s = lax.axis_index("s")
    w = c * NS + s
    zero16 = jnp.zeros((16,), jnp.float32)
    ones16 = jnp.ones((16,), jnp.float32)

    pltpu.sync_copy(edge_hbm.at[w], dstv)

    def zbody(i, _):
      zbuf[i, :] = zero16
      return 0
    lax.fori_loop(0, STRIPE, zbody, 0)
    pltpu.sync_copy(zbuf, acc.at[pl.ds(s * STRIPE, STRIPE), :])

    def obody(i, _):
      ones_rows[i, :] = ones16
      return 0
    lax.fori_loop(0, CH, obody, 0)
    plsc.subcore_barrier()

    sd = [None] * NCHT
    for j in range(NCHT):
      if j >= NB:
        sd[j - NB].wait()
      sd[j] = pltpu.async_copy(ones_rows, acc.at[dstv.at[j, 1]],
                               ssem[j % NB], add=True)
    for j in range(max(0, NCHT - NB), NCHT):
      sd[j].wait()
    plsc.subcore_barrier()
    pltpu.sync_copy(acc.at[pl.ds(s * STRIPE, STRIPE), :],
                    cnt_hbm.at[c, pl.ds(s * STRIPE, STRIPE), :])

  return count


def _make_hop_scatter(NW, NCHT, N_pad, F_OUT):
  NC, NS = _sc_geometry()
  mesh = plsc.VectorSubcoreMesh(core_axis_name="c", subcore_axis_name="s")
  STRIPE = N_pad // NS

  NB = 3

  @functools.partial(
      pl.kernel,
      out_type=jax.ShapeDtypeStruct((NC, N_pad, F_OUT), jnp.float32),
      mesh=mesh,
      scratch_types=[pltpu.VMEM_SHARED((N_pad, F_OUT), jnp.float32)]
      + [pltpu.VMEM((2, CH), jnp.int32) for _ in range(NB)]
      + [pltpu.VMEM((CH, F_OUT), jnp.float32) for _ in range(NB)]
      + [pltpu.SemaphoreType.DMA for _ in range(3 * NB)],
  )
  def hop(g_hbm, edge_hbm, s_hbm, acc, *scr):
    idxb = scr[:NB]
    rows = scr[NB:2 * NB]
    isem = scr[2 * NB:3 * NB]
    gsem = scr[3 * NB:4 * NB]
    ssem = scr[4 * NB:]
    c = lax.axis_index("c")
    s = lax.axis_index("s")
    w = c * NS + s
    zero16 = jnp.zeros((16,), jnp.float32)

    idxd = [None] * NCHT
    for j in range(min(NB, NCHT)):
      idxd[j] = pltpu.async_copy(edge_hbm.at[w, j], idxb[j % NB], isem[j % NB])

    def zbody(i, _):
      for k in range(F_OUT // 16):
        rows[0][i, pl.ds(k * 16, 16)] = zero16
      return 0
    lax.fori_loop(0, CH, zbody, 0)
    nfull, rem = STRIPE // CH, STRIPE % CH
    for i in range(nfull):
      pltpu.sync_copy(rows[0], acc.at[pl.ds(s * STRIPE + i * CH, CH), :])
    if rem:
      pltpu.sync_copy(rows[0].at[pl.ds(0, rem), :],
                      acc.at[pl.ds(s * STRIPE + nfull * CH, rem), :])
    plsc.subcore_barrier()

    gd = [None] * NCHT
    sd = [None] * NCHT
    for j in range(NCHT):
      b = j % NB
      if j >= NB:
        sd[j - NB].wait()
        idxd[j] = pltpu.async_copy(edge_hbm.at[w, j], idxb[b], isem[b])
      idxd[j].wait()
      gd[j] = pltpu.async_copy(g_hbm.at[idxb[b].at[0]], rows[b], gsem[b])
      if j >= NB - 1:
        k = j - (NB - 1)
        gd[k].wait()
        sd[k] = pltpu.async_copy(rows[k % NB], acc.at[idxb[k % NB].at[1]],
                                 ssem[k % NB], add=True)
    for k in range(max(0, NCHT - (NB - 1)), NCHT):
      gd[k].wait()
      sd[k] = pltpu.async_copy(rows[k % NB], acc.at[idxb[k % NB].at[1]],
                               ssem[k % NB], add=True)
    for k in range(max(0, NCHT - NB), NCHT):
      sd[k].wait()

    plsc.subcore_barrier()
    pltpu.sync_copy(acc.at[pl.ds(s * STRIPE, STRIPE), :],
                    s_hbm.at[c, pl.ds(s * STRIPE, STRIPE), :])

  return hop


def _mlp_body(x_ref, w1_ref, b1_ref, w2_ref, b2_ref, h0_ref):
  t = jnp.dot(x_ref[...], w1_ref[...], preferred_element_type=jnp.float32)
  t = jnp.maximum(t + b1_ref[...], 0.0)
  h0_ref[...] = (
      jnp.dot(t, w2_ref[...], preferred_element_type=jnp.float32) + b2_ref[...]
  )


def _prep_body(cnt_ref, h0_ref, g0_ref, dinv_ref):
  deg = 1.0 + jnp.sum(cnt_ref[...], axis=(0, 2)) * (1.0 / DW)
  dinv = lax.rsqrt(deg)[:, None]
  dinv_ref[...] = dinv
  g0_ref[...] = h0_ref[...] * dinv


def _update_body(s_ref, g_ref, h0_ref, dinv_ref, gn_ref):
  agg = s_ref[0] + s_ref[1] + g_ref[...]
  dinv = dinv_ref[...]
  h_new = ALPHA * h0_ref[...] + (1.0 - ALPHA) * (dinv * agg)
  gn_ref[...] = dinv * h_new


def _update_last_body(s_ref, g_ref, h0_ref, dinv_ref, out_ref):
  agg = s_ref[0] + s_ref[1] + g_ref[...]
  dinv = dinv_ref[...]
  h_new = ALPHA * h0_ref[...] + (1.0 - ALPHA) * (dinv * agg)
  m = jnp.max(h_new, axis=1, keepdims=True)
  z = h_new - m
  lse = jnp.log(jnp.sum(jnp.exp(z), axis=1, keepdims=True))
  out_ref[...] = z - lse


def kernel(x, edge_index, W1, b1, W2, b2):
  N, F_IN = x.shape
  F_HID = W1.shape[1]
  F_OUT = W2.shape[1]
  E = edge_index.shape[1]
  NC, NS = _sc_geometry()
  NW = NC * NS

  N_pad = ((N + 1 + ROWB - 1) // ROWB) * ROWB
  NCHT = -(-E // (NW * CH))
  E_pad = NW * CH * NCHT

  src = edge_index[0].astype(jnp.int32)
  dst = edge_index[1].astype(jnp.int32)
  fill = N + jnp.arange(E_pad - E, dtype=jnp.int32) % (N_pad - N)
  edges_p = jnp.stack([
      jnp.concatenate([src, fill]),
      jnp.concatenate([dst, fill]),
  ]).reshape(2, NW, NCHT, CH).transpose(1, 2, 0, 3)
  x_p = jnp.zeros((N_pad, F_IN), x.dtype).at[:N].set(x)
  b1r = b1.reshape(1, F_HID)
  b2r = b2.reshape(1, F_OUT)

  grid = (N_pad // ROWB,)

  cnt = _make_count(NW, NCHT, N_pad)(edges_p)

  h0 = pl.pallas_call(
      _mlp_body,
      grid=grid,
      in_specs=[
          pl.BlockSpec((ROWB, F_IN), lambda i: (i, 0)),
          pl.BlockSpec((F_IN, F_HID), lambda i: (0, 0)),
          pl.BlockSpec((1, F_HID), lambda i: (0, 0)),
          pl.BlockSpec((F_HID, F_OUT), lambda i: (0, 0)),
          pl.BlockSpec((1, F_OUT), lambda i: (0, 0)),
      ],
      out_specs=pl.BlockSpec((ROWB, F_OUT), lambda i: (i, 0)),
      out_shape=jax.ShapeDtypeStruct((N_pad, F_OUT), jnp.float32),
  )(x_p, W1, b1r, W2, b2r)

  g, dinv = pl.pallas_call(
      _prep_body,
      grid=grid,
      in_specs=[
          pl.BlockSpec((NC, ROWB, DW), lambda i: (0, i, 0)),
          pl.BlockSpec((ROWB, F_OUT), lambda i: (i, 0)),
      ],
      out_specs=[
          pl.BlockSpec((ROWB, F_OUT), lambda i: (i, 0)),
          pl.BlockSpec((ROWB, 1), lambda i: (i, 0)),
      ],
      out_shape=[
          jax.ShapeDtypeStruct((N_pad, F_OUT), jnp.float32),
          jax.ShapeDtypeStruct((N_pad, 1), jnp.float32),
      ],
  )(cnt, h0)

  hop = _make_hop_scatter(NW, NCHT, N_pad, F_OUT)
  upd_specs = dict(
      grid=grid,
      in_specs=[
          pl.BlockSpec((NC, ROWB, F_OUT), lambda i: (0, i, 0)),
          pl.BlockSpec((ROWB, F_OUT), lambda i: (i, 0)),
          pl.BlockSpec((ROWB, F_OUT), lambda i: (i, 0)),
          pl.BlockSpec((ROWB, 1), lambda i: (i, 0)),
      ],
      out_specs=pl.BlockSpec((ROWB, F_OUT), lambda i: (i, 0)),
      out_shape=jax.ShapeDtypeStruct((N_pad, F_OUT), jnp.float32),
  )

  for k in range(K_HOPS):
    S = hop(g, edges_p)
    body = _update_last_body if k == K_HOPS - 1 else _update_body
    g = pl.pallas_call(body, **upd_specs)(S, g, h0, dinv)

  return g[:N]


# speedup vs baseline: 15.4265x; 1.3927x over previous
import functools

import jax
import jax.numpy as jnp
from jax import lax
from jax.experimental import pallas as pl
from jax.experimental.pallas import tpu as pltpu
from jax.experimental.pallas import tpu_sc as plsc

K_HOPS = 10
ALPHA = 0.1
CH = 128
ROWB = 128


def _sc_geometry():
  info = plsc.get_sparse_core_info()
  return info.num_cores, info.num_subcores


DW = 16


def _make_count(NW, NCHT, N_pad):
  NC, NS = _sc_geometry()
  mesh = plsc.VectorSubcoreMesh(core_axis_name="c", subcore_axis_name="s")
  STRIPE = N_pad // NS

  NB = 4

  @functools.partial(
      pl.kernel,
      out_type=jax.ShapeDtypeStruct((NC, N_pad, DW), jnp.float32),
      mesh=mesh,
      scratch_types=[
          pltpu.VMEM_SHARED((N_pad, DW), jnp.float32),
          pltpu.VMEM((STRIPE, DW), jnp.float32),
          pltpu.VMEM((NCHT, 2, CH), jnp.int32),
          pltpu.VMEM((CH, DW), jnp.float32),
      ] + [pltpu.SemaphoreType.DMA for _ in range(NB)],
  )
  def count(edge_hbm, cnt_hbm, acc, zbuf, dstv, ones_rows, *ssem):
    c = lax.axis_index("c")
    s = lax.axis_index("s")
    w = c * NS + s
    zero16 = jnp.zeros((16,), jnp.float32)
    ones16 = jnp.ones((16,), jnp.float32)

    pltpu.sync_copy(edge_hbm.at[w], dstv)

    def zbody(i, _):
      zbuf[i, :] = zero16
      return 0
    lax.fori_loop(0, STRIPE, zbody, 0)
    pltpu.sync_copy(zbuf, acc.at[pl.ds(s * STRIPE, STRIPE), :])

    def obody(i, _):
      ones_rows[i, :] = ones16
      return 0
    lax.fori_loop(0, CH, obody, 0)
    plsc.subcore_barrier()

    sd = [None] * NCHT
    for j in range(NCHT):
      if j >= NB:
        sd[j - NB].wait()
      sd[j] = pltpu.async_copy(ones_rows, acc.at[dstv.at[j, 1]],
                               ssem[j % NB], add=True)
    for j in range(max(0, NCHT - NB), NCHT):
      sd[j].wait()
    plsc.subcore_barrier()
    pltpu.sync_copy(acc.at[pl.ds(s * STRIPE, STRIPE), :],
                    cnt_hbm.at[c, pl.ds(s * STRIPE, STRIPE), :])

  return count


def _make_hop_scatter(NW, NCHT, N_pad, F_OUT):
  NC, NS = _sc_geometry()
  mesh = plsc.VectorSubcoreMesh(core_axis_name="c", subcore_axis_name="s")
  STRIPE = N_pad // NS

  NB = 3

  @functools.partial(
      pl.kernel,
      out_type=jax.ShapeDtypeStruct((NC, N_pad, F_OUT), jnp.float32),
      mesh=mesh,
      scratch_types=[pltpu.VMEM_SHARED((N_pad, F_OUT), jnp.float32)]
      + [pltpu.VMEM((2, CH), jnp.int32) for _ in range(NB)]
      + [pltpu.VMEM((CH, F_OUT), jnp.float32) for _ in range(NB)]
      + [pltpu.SemaphoreType.DMA for _ in range(3 * NB)],
  )
  def hop(g_hbm, edge_hbm, s_hbm, acc, *scr):
    idxb = scr[:NB]
    rows = scr[NB:2 * NB]
    isem = scr[2 * NB:3 * NB]
    gsem = scr[3 * NB:4 * NB]
    ssem = scr[4 * NB:]
    c = lax.axis_index("c")
    s = lax.axis_index("s")
    w = c * NS + s
    zero16 = jnp.zeros((16,), jnp.float32)

    idxd = [None] * NCHT
    for j in range(min(NB, NCHT)):
      idxd[j] = pltpu.async_copy(edge_hbm.at[w, j], idxb[j % NB], isem[j % NB])

    def zbody(i, _):
      for k in range(F_OUT // 16):
        rows[0][i, pl.ds(k * 16, 16)] = zero16
      return 0
    lax.fori_loop(0, CH, zbody, 0)
    nfull, rem = STRIPE // CH, STRIPE % CH
    for i in range(nfull):
      pltpu.sync_copy(rows[0], acc.at[pl.ds(s * STRIPE + i * CH, CH), :])
    if rem:
      pltpu.sync_copy(rows[0].at[pl.ds(0, rem), :],
                      acc.at[pl.ds(s * STRIPE + nfull * CH, rem), :])
    plsc.subcore_barrier()

    gd = [None] * NCHT
    sd = [None] * NCHT
    for j in range(NCHT):
      b = j % NB
      if j >= NB:
        sd[j - NB].wait()
        idxd[j] = pltpu.async_copy(edge_hbm.at[w, j], idxb[b], isem[b])
      idxd[j].wait()
      gd[j] = pltpu.async_copy(g_hbm.at[idxb[b].at[0]], rows[b], gsem[b])
      if j >= NB - 1:
        k = j - (NB - 1)
        gd[k].wait()
        sd[k] = pltpu.async_copy(rows[k % NB], acc.at[idxb[k % NB].at[1]],
                                 ssem[k % NB], add=True)
    for k in range(max(0, NCHT - (NB - 1)), NCHT):
      gd[k].wait()
      sd[k] = pltpu.async_copy(rows[k % NB], acc.at[idxb[k % NB].at[1]],
                               ssem[k % NB], add=True)
    for k in range(max(0, NCHT - NB), NCHT):
      sd[k].wait()

    plsc.subcore_barrier()
    pltpu.sync_copy(acc.at[pl.ds(s * STRIPE, STRIPE), :],
                    s_hbm.at[c, pl.ds(s * STRIPE, STRIPE), :])

  return hop


def _mlp_body(x_ref, w1_ref, b1_ref, w2_ref, b2_ref, h0_ref):
  t = jnp.dot(x_ref[...], w1_ref[...], preferred_element_type=jnp.float32)
  t = jnp.maximum(t + b1_ref[...], 0.0)
  h0_ref[...] = (
      jnp.dot(t, w2_ref[...], preferred_element_type=jnp.float32) + b2_ref[...]
  )


def _prep_body(cnt_ref, h0_ref, g0_ref, dinv_ref):
  deg = 1.0 + jnp.sum(cnt_ref[...], axis=(0, 2)) * (1.0 / DW)
  dinv = lax.rsqrt(deg)[:, None]
  dinv_ref[...] = dinv
  g0_ref[...] = h0_ref[...] * dinv


def _update_body(s_ref, g_ref, h0_ref, dinv_ref, gn_ref):
  agg = s_ref[0] + s_ref[1] + g_ref[...]
  dinv = dinv_ref[...]
  h_new = ALPHA * h0_ref[...] + (1.0 - ALPHA) * (dinv * agg)
  gn_ref[...] = dinv * h_new


def _update_last_body(s_ref, g_ref, h0_ref, dinv_ref, out_ref):
  agg = s_ref[0] + s_ref[1] + g_ref[...]
  dinv = dinv_ref[...]
  h_new = ALPHA * h0_ref[...] + (1.0 - ALPHA) * (dinv * agg)
  m = jnp.max(h_new, axis=1, keepdims=True)
  z = h_new - m
  lse = jnp.log(jnp.sum(jnp.exp(z), axis=1, keepdims=True))
  out_ref[...] = z - lse


def kernel(x, edge_index, W1, b1, W2, b2):
  N, F_IN = x.shape
  F_HID = W1.shape[1]
  F_OUT = W2.shape[1]
  E = edge_index.shape[1]
  NC, NS = _sc_geometry()
  NW = NC * NS

  N_pad = ((N + 1 + ROWB - 1) // ROWB) * ROWB
  NCHT = -(-E // (NW * CH))
  E_pad = NW * CH * NCHT

  src = edge_index[0].astype(jnp.int32)
  dst = edge_index[1].astype(jnp.int32)
  fill = N + jnp.arange(E_pad - E, dtype=jnp.int32) % (N_pad - N)
  edges_p = jnp.stack([
      jnp.concatenate([src, fill]),
      jnp.concatenate([dst, fill]),
  ]).reshape(2, NW, NCHT, CH).transpose(1, 2, 0, 3)
  x_p = jnp.zeros((N_pad, F_IN), x.dtype).at[:N].set(x)
  b1r = b1.reshape(1, F_HID)
  b2r = b2.reshape(1, F_OUT)

  grid = (N_pad // ROWB,)

  cnt = _make_count(NW, NCHT, N_pad)(edges_p)

  h0 = pl.pallas_call(
      _mlp_body,
      grid=grid,
      in_specs=[
          pl.BlockSpec((ROWB, F_IN), lambda i: (i, 0)),
          pl.BlockSpec((F_IN, F_HID), lambda i: (0, 0)),
          pl.BlockSpec((1, F_HID), lambda i: (0, 0)),
          pl.BlockSpec((F_HID, F_OUT), lambda i: (0, 0)),
          pl.BlockSpec((1, F_OUT), lambda i: (0, 0)),
      ],
      out_specs=pl.BlockSpec((ROWB, F_OUT), lambda i: (i, 0)),
      out_shape=jax.ShapeDtypeStruct((N_pad, F_OUT), jnp.float32),
  )(x_p, W1, b1r, W2, b2r)

  g, dinv = pl.pallas_call(
      _prep_body,
      grid=grid,
      in_specs=[
          pl.BlockSpec((NC, ROWB, DW), lambda i: (0, i, 0)),
          pl.BlockSpec((ROWB, F_OUT), lambda i: (i, 0)),
      ],
      out_specs=[
          pl.BlockSpec((ROWB, F_OUT), lambda i: (i, 0)),
          pl.BlockSpec((ROWB, 1), lambda i: (i, 0)),
      ],
      out_shape=[
          jax.ShapeDtypeStruct((N_pad, F_OUT), jnp.float32),
          jax.ShapeDtypeStruct((N_pad, 1), jnp.float32),
      ],
  )(cnt, h0)

  hop = _make_hop_scatter(NW, NCHT, N_pad, F_OUT)
  UB = ROWB
  for cand in (2048, 1792, 1536, 1280, 1264, 1024, 896, 768, 640, 512, 384, 256):
    if N_pad % cand == 0 and cand % 8 == 0:
      UB = cand
      break
  upd_specs = dict(
      grid=(N_pad // UB,),
      in_specs=[
          pl.BlockSpec((NC, UB, F_OUT), lambda i: (0, i, 0)),
          pl.BlockSpec((UB, F_OUT), lambda i: (i, 0)),
          pl.BlockSpec((UB, F_OUT), lambda i: (i, 0)),
          pl.BlockSpec((UB, 1), lambda i: (i, 0)),
      ],
      out_specs=pl.BlockSpec((UB, F_OUT), lambda i: (i, 0)),
      out_shape=jax.ShapeDtypeStruct((N_pad, F_OUT), jnp.float32),
  )

  for k in range(K_HOPS):
    S = hop(g, edges_p)
    body = _update_last_body if k == K_HOPS - 1 else _update_body
    g = pl.pallas_call(body, **upd_specs)(S, g, h0, dinv)

  return g[:N]
